# baseline (speedup 1.0000x reference)
import jax
import jax.numpy as jnp
from jax import lax
from jax.experimental import pallas as pl
from jax.experimental.pallas import tpu as pltpu
from jax.experimental.pallas import tpu_sc as plsc

B, C, H, W = 32, 384, 28, 28
N = B * C * H * W
BLK = 3 * 8 * 128
NBLK = H * W * (B // 8)
NW = 32
WBLK = NBLK // NW
CHUNK_B = 7
S = CHUNK_B * BLK
NCHUNK = WBLK // CHUNK_B
LANES = 16
NBUF = 2

_C0S = (0, 16, 32, 48, 64, 80)
_OFFS = tuple(
    tuple((ch // 128) * 1024 + ch % 128 for ch in (c0, c0 + 96, c0 + 192, c0 + 288))
    for c0 in _C0S
)

_mesh = plsc.VectorSubcoreMesh(core_axis_name="c", subcore_axis_name="s")


@pl.kernel(
    out_type=jax.ShapeDtypeStruct((N,), jnp.float32),
    mesh=_mesh,
    scratch_types=[
        [pltpu.VMEM((S,), jnp.float32) for _ in range(NBUF)],
        [pltpu.VMEM((S,), jnp.float32) for _ in range(NBUF)],
        [pltpu.SemaphoreType.DMA for _ in range(NBUF)],
        [pltpu.SemaphoreType.DMA for _ in range(NBUF)],
    ],
)
def _topk_sc(x_hbm, out_hbm, inb, outb, insem, outsem):
    wid = lax.axis_index("s") * 2 + lax.axis_index("c")
    base = wid * (WBLK * BLK)

    def load(t, slot):
        return pltpu.async_copy(
            x_hbm.at[pl.ds(base + t * S, S)], inb[slot], insem[slot]
        )

    def store(t, slot):
        return pltpu.async_copy(
            outb[slot], out_hbm.at[pl.ds(base + t * S, S)], outsem[slot]
        )

    in_cp = load(0, 0)
    out_cps = [None] * NBUF
    for t in range(NCHUNK):
        slot = t % NBUF
        if t + 1 < NCHUNK:
            next_cp = load(t + 1, (t + 1) % NBUF)
        in_cp.wait()
        if out_cps[slot] is not None:
            out_cps[slot].wait()
            out_cps[slot] = None

        src = inb[slot]
        dst = outb[slot]

        @plsc.parallel_loop(0, CHUNK_B * 8, unroll=2)
        def _network(r):
            row = (r // 8) * BLK + (r % 8) * 128
            for offs in _OFFS:
                s0 = pl.ds(row + offs[0], LANES)
                s1 = pl.ds(row + offs[1], LANES)
                s2 = pl.ds(row + offs[2], LANES)
                s3 = pl.ds(row + offs[3], LANES)
                v0 = src[s0]
                v1 = src[s1]
                v2 = src[s2]
                v3 = src[s3]
                l1 = jnp.minimum(v0, v1)
                h1 = jnp.maximum(v0, v1)
                l2 = jnp.minimum(v2, v3)
                h2 = jnp.maximum(v2, v3)
                m1 = jnp.minimum(h1, h2)
                m2 = jnp.maximum(l1, l2)
                dst[s0] = jnp.maximum(h1, h2)
                dst[s1] = jnp.maximum(m1, m2)
                dst[s2] = jnp.minimum(m1, m2)
                dst[s3] = jnp.minimum(l1, l2)

        out_cps[slot] = store(t, slot)
        if t + 1 < NCHUNK:
            in_cp = next_cp
    for cp in out_cps:
        if cp is not None:
            cp.wait()


def kernel(x):
    z = x.reshape(4, 8, 3, 128, H, W).transpose(4, 5, 0, 2, 1, 3).reshape(N)
    o = _topk_sc(z)
    return (
        o.reshape(H, W, 4, 3, 8, 128)
        .transpose(2, 4, 3, 5, 0, 1)
        .reshape(B, C, H, W)
    )


# speedup vs baseline: 1.0429x; 1.0429x over previous
import jax
import jax.numpy as jnp
from jax import lax
from jax.experimental import pallas as pl
from jax.experimental.pallas import tpu as pltpu
from jax.experimental.pallas import tpu_sc as plsc

B, C, H, W = 32, 384, 28, 28
N = B * C * H * W
BLK = 3 * 8 * 128
NBLK = H * W * (B // 8)
NW = 32
WBLK = NBLK // NW
CHUNK_B = 10
_CHUNKS = [10] * 9 + [8]
_STARTS = [sum(_CHUNKS[:i]) for i in range(len(_CHUNKS))]
S = CHUNK_B * BLK
NCHUNK = len(_CHUNKS)
LANES = 16
NBUF = 2

_C0S = (0, 16, 32, 48, 64, 80)
_OFFS = tuple(
    tuple((ch // 128) * 1024 + ch % 128 for ch in (c0, c0 + 96, c0 + 192, c0 + 288))
    for c0 in _C0S
)

_mesh = plsc.VectorSubcoreMesh(core_axis_name="c", subcore_axis_name="s")


@pl.kernel(
    out_type=jax.ShapeDtypeStruct((N,), jnp.float32),
    mesh=_mesh,
    scratch_types=[
        [pltpu.VMEM((S,), jnp.float32) for _ in range(NBUF)],
        [pltpu.VMEM((S,), jnp.float32) for _ in range(NBUF)],
        [pltpu.SemaphoreType.DMA for _ in range(NBUF)],
        [pltpu.SemaphoreType.DMA for _ in range(NBUF)],
    ],
)
def _topk_sc(x_hbm, out_hbm, inb, outb, insem, outsem):
    wid = lax.axis_index("s") * 2 + lax.axis_index("c")
    base = wid * (WBLK * BLK)

    def load(t, slot):
        n = _CHUNKS[t] * BLK
        return pltpu.async_copy(
            x_hbm.at[pl.ds(base + _STARTS[t] * BLK, n)],
            inb[slot].at[pl.ds(0, n)],
            insem[slot],
        )

    def store(t, slot):
        n = _CHUNKS[t] * BLK
        return pltpu.async_copy(
            outb[slot].at[pl.ds(0, n)],
            out_hbm.at[pl.ds(base + _STARTS[t] * BLK, n)],
            outsem[slot],
        )

    in_cp = load(0, 0)
    out_cps = [None] * NBUF
    for t in range(NCHUNK):
        slot = t % NBUF
        if t + 1 < NCHUNK:
            next_cp = load(t + 1, (t + 1) % NBUF)
        in_cp.wait()
        if out_cps[slot] is not None:
            out_cps[slot].wait()
            out_cps[slot] = None

        src = inb[slot]
        dst = outb[slot]

        @plsc.parallel_loop(0, _CHUNKS[t] * 8)
        def _network(r):
            row = (r // 8) * BLK + (r % 8) * 128
            for offs in _OFFS:
                s0 = pl.ds(row + offs[0], LANES)
                s1 = pl.ds(row + offs[1], LANES)
                s2 = pl.ds(row + offs[2], LANES)
                s3 = pl.ds(row + offs[3], LANES)
                v0 = src[s0]
                v1 = src[s1]
                v2 = src[s2]
                v3 = src[s3]
                l1 = jnp.minimum(v0, v1)
                h1 = jnp.maximum(v0, v1)
                l2 = jnp.minimum(v2, v3)
                h2 = jnp.maximum(v2, v3)
                m1 = jnp.minimum(h1, h2)
                m2 = jnp.maximum(l1, l2)
                dst[s0] = jnp.maximum(h1, h2)
                dst[s1] = jnp.maximum(m1, m2)
                dst[s2] = jnp.minimum(m1, m2)
                dst[s3] = jnp.minimum(l1, l2)

        out_cps[slot] = store(t, slot)
        if t + 1 < NCHUNK:
            in_cp = next_cp
    for cp in out_cps:
        if cp is not None:
            cp.wait()


def kernel(x):
    z = x.reshape(4, 8, 3, 128, H, W).transpose(4, 5, 0, 2, 1, 3).reshape(N)
    o = _topk_sc(z)
    return (
        o.reshape(H, W, 4, 3, 8, 128)
        .transpose(2, 4, 3, 5, 0, 1)
        .reshape(B, C, H, W)
    )
